# Initial kernel scaffold; baseline (speedup 1.0000x reference)
import jax
import jax.numpy as jnp
from jax.experimental import pallas as pl


def kernel(user_ids, item_ids, text_features, image_features, edge_index, params):
    raise NotImplementedError("write your pallas kernel here")


# speedup vs baseline: 1.0704x; 1.0704x over previous
import functools

import jax
import jax.numpy as jnp
from jax.experimental import pallas as pl

HIDDEN = 128
GAT_HEADS = 4
NUM_LAYERS = 3


def _feat_body(t_ref, im_ref, wt_ref, wi_ref, b_ref, o_ref):
    acc = jnp.dot(t_ref[...], wt_ref[...], preferred_element_type=jnp.float32)
    acc += jnp.dot(im_ref[...], wi_ref[...], preferred_element_type=jnp.float32)
    o_ref[...] = acc + b_ref[...]


def _feat_matmul(text, image, W_t, W_i, bias):
    B = text.shape[0]
    BLK = 1024
    grid = (B // BLK,)
    return pl.pallas_call(
        _feat_body,
        grid=grid,
        in_specs=[
            pl.BlockSpec((BLK, text.shape[1]), lambda i: (i, 0)),
            pl.BlockSpec((BLK, image.shape[1]), lambda i: (i, 0)),
            pl.BlockSpec(W_t.shape, lambda i: (0, 0)),
            pl.BlockSpec(W_i.shape, lambda i: (0, 0)),
            pl.BlockSpec(bias.shape, lambda i: (0,)),
        ],
        out_specs=pl.BlockSpec((BLK, W_t.shape[1]), lambda i: (i, 0)),
        out_shape=jax.ShapeDtypeStruct((B, W_t.shape[1]), jnp.float32),
    )(text, image, W_t, W_i, bias)


def _gat(x, src, dst, g):
    n = x.shape[0]
    h = (x @ g['W']).reshape(n, GAT_HEADS, -1)
    a_src = jnp.sum(h * g['att_src'][None], axis=-1)
    a_dst = jnp.sum(h * g['att_dst'][None], axis=-1)
    e = a_src[src] + a_dst[dst]
    e = jax.nn.leaky_relu(e, 0.2)
    w = jnp.exp(e)
    denom = jax.ops.segment_sum(w, dst, num_segments=n)
    msg = h[src] * w[:, :, None]
    out = jax.ops.segment_sum(msg, dst, num_segments=n)
    out = out / (denom[:, :, None] + 1e-16)
    return out.reshape(n, -1) + g['b']


def _ln(x, g, b):
    m = jnp.mean(x, axis=-1, keepdims=True)
    v = jnp.var(x, axis=-1, keepdims=True)
    return (x - m) / jnp.sqrt(v + 1e-5) * g + b


def kernel(user_ids, item_ids, text_features, image_features, edge_index, params):
    p = params
    bsz = user_ids.shape[0]
    tn = text_features.shape[0]
    imn = image_features.shape[0]

    M_u = p['ua']['Wv'] @ p['ua']['Wo']
    c_u = p['ua']['bv'] @ p['ua']['Wo'] + p['ua']['bo']
    M_i = p['ia']['Wv'] @ p['ia']['Wo']
    c_i = p['ia']['bv'] @ p['ia']['Wo'] + p['ia']['bo']
    half = HIDDEN // 2
    A_u = p['ut_W'] @ M_u[:half]
    B_u = p['ui_W'] @ M_u[half:]
    b_u = p['ut_b'] @ M_u[:half] + p['ui_b'] @ M_u[half:] + c_u
    A_i = p['it_W'] @ M_i[:half]
    B_i = p['ii_W'] @ M_i[half:]
    b_i = p['it_b'] @ M_i[:half] + p['ii_b'] @ M_i[half:] + c_i

    W_t = jnp.concatenate([A_u, A_i], axis=1)
    W_im = jnp.concatenate([B_u, B_i], axis=1)
    bias = jnp.concatenate([b_u, b_i], axis=0)

    P = _feat_matmul(text_features, image_features, W_t, W_im, bias)
    UM_all = P[:, :HIDDEN]
    IM_all = P[:, HIDDEN:]

    user_emb = jnp.take(p['user_table'], user_ids, axis=0)
    item_emb = jnp.take(p['item_table'], item_ids, axis=0)
    um = jnp.take(UM_all, user_ids % tn, axis=0)
    im = jnp.take(IM_all, item_ids % imn, axis=0)

    x = jnp.concatenate([
        jnp.concatenate([user_emb, um], axis=1),
        jnp.concatenate([item_emb, im], axis=1),
    ], axis=0)

    src = edge_index[0]
    dst = edge_index[1]
    for i in range(NUM_LAYERS):
        if i > 0:
            x = x + (x @ p['res%d_W' % (i - 1)] + p['res%d_b' % (i - 1)])
        x = _gat(x, src, dst, p['gat%d' % i])
        x = _ln(x, p['ln%d_g' % i], p['ln%d_b' % i])
        x = jax.nn.relu(x)

    ue = x[:bsz]
    ie = x[bsz:2 * bsz]
    scores = jnp.sum(ue * ie, axis=1)
    return scores, ue, ie


# speedup vs baseline: 27.8600x; 26.0274x over previous
import functools

import jax
import jax.numpy as jnp
from jax import lax
from jax.experimental import pallas as pl
from jax.experimental.pallas import tpu as pltpu
from jax.experimental.pallas import tpu_sc as plsc

HIDDEN = 128
HEADS = 4
HEAD_DIM = 32
NUM_LAYERS = 3
ACCW = 40
GROUP = 512
NC = 2
NS = 16

_MESH = plsc.VectorSubcoreMesh(core_axis_name="c", subcore_axis_name="s")
_SC_PARAMS = pltpu.CompilerParams(needs_layout_passes=False)
_SC_PARAMS_UNTILED = pltpu.CompilerParams(
    needs_layout_passes=False, use_tc_tiling_on_sc=False)


def _feat_body(t_ref, im_ref, wt_ref, wi_ref, b_ref, o_ref):
    acc = jnp.dot(t_ref[...], wt_ref[...], preferred_element_type=jnp.float32)
    acc += jnp.dot(im_ref[...], wi_ref[...], preferred_element_type=jnp.float32)
    o_ref[...] = acc + b_ref[...]


def _feat_matmul(text, image, W_t, W_i, bias):
    B = text.shape[0]
    BLK = 1024
    return pl.pallas_call(
        _feat_body,
        grid=(B // BLK,),
        in_specs=[
            pl.BlockSpec((BLK, text.shape[1]), lambda i: (i, 0)),
            pl.BlockSpec((BLK, image.shape[1]), lambda i: (i, 0)),
            pl.BlockSpec(W_t.shape, lambda i: (0, 0)),
            pl.BlockSpec(W_i.shape, lambda i: (0, 0)),
            pl.BlockSpec(bias.shape, lambda i: (0,)),
        ],
        out_specs=pl.BlockSpec((BLK, W_t.shape[1]), lambda i: (i, 0)),
        out_shape=jax.ShapeDtypeStruct((B, W_t.shape[1]), jnp.float32),
    )(text, image, W_t, W_i, bias)


def _pre0_body(xe_ref, xm_ref, w1_ref, w2_ref, att_ref, hh_ref, a8_ref):
    h = jnp.dot(xe_ref[...], w1_ref[...], preferred_element_type=jnp.float32)
    h += jnp.dot(xm_ref[...], w2_ref[...], preferred_element_type=jnp.float32)
    for k in range(HEADS):
        hh_ref[k] = h[:, k * HEAD_DIM:(k + 1) * HEAD_DIM]
    a8_ref[...] = jnp.dot(h, att_ref[...], preferred_element_type=jnp.float32)


def _layer_pre0(x_emb, x_mm, W1, W2, attm):
    N = x_emb.shape[0]
    BLK = 2048
    return pl.pallas_call(
        _pre0_body,
        grid=(N // BLK,),
        in_specs=[
            pl.BlockSpec((BLK, HIDDEN), lambda i: (i, 0)),
            pl.BlockSpec((BLK, HIDDEN), lambda i: (i, 0)),
            pl.BlockSpec((HIDDEN, HIDDEN), lambda i: (0, 0)),
            pl.BlockSpec((HIDDEN, HIDDEN), lambda i: (0, 0)),
            pl.BlockSpec((HIDDEN, 2 * HEADS), lambda i: (0, 0)),
        ],
        out_specs=[
            pl.BlockSpec((HEADS, BLK, HEAD_DIM), lambda i: (0, i, 0)),
            pl.BlockSpec((BLK, 2 * HEADS), lambda i: (i, 0)),
        ],
        out_shape=[
            jax.ShapeDtypeStruct((HEADS, N, HEAD_DIM), jnp.float32),
            jax.ShapeDtypeStruct((N, 2 * HEADS), jnp.float32),
        ],
    )(x_emb, x_mm, W1, W2, attm)


def _pre_body(x_ref, rw_ref, rb_ref, w_ref, att_ref, hh_ref, a8_ref):
    x = x_ref[...]
    xr = x + jnp.dot(x, rw_ref[...], preferred_element_type=jnp.float32) + rb_ref[...]
    h = jnp.dot(xr, w_ref[...], preferred_element_type=jnp.float32)
    for k in range(HEADS):
        hh_ref[k] = h[:, k * HEAD_DIM:(k + 1) * HEAD_DIM]
    a8_ref[...] = jnp.dot(h, att_ref[...], preferred_element_type=jnp.float32)


def _layer_pre(x, resW, resb, W, attm):
    N = x.shape[0]
    BLK = 2048
    return pl.pallas_call(
        _pre_body,
        grid=(N // BLK,),
        in_specs=[
            pl.BlockSpec((BLK, HIDDEN), lambda i: (i, 0)),
            pl.BlockSpec((HIDDEN, HIDDEN), lambda i: (0, 0)),
            pl.BlockSpec((HIDDEN,), lambda i: (0,)),
            pl.BlockSpec((HIDDEN, HIDDEN), lambda i: (0, 0)),
            pl.BlockSpec((HIDDEN, 2 * HEADS), lambda i: (0, 0)),
        ],
        out_specs=[
            pl.BlockSpec((HEADS, BLK, HEAD_DIM), lambda i: (0, i, 0)),
            pl.BlockSpec((BLK, 2 * HEADS), lambda i: (i, 0)),
        ],
        out_shape=[
            jax.ShapeDtypeStruct((HEADS, N, HEAD_DIM), jnp.float32),
            jax.ShapeDtypeStruct((N, 2 * HEADS), jnp.float32),
        ],
    )(x, resW, resb, W, attm)


def _fin_body(acc_ref, b_ref, g_ref, lb_ref, o_ref):
    cols = []
    for k in range(HEADS):
        a = acc_ref[k]
        num = a[:, :HEAD_DIM]
        den = a[:, HEAD_DIM:HEAD_DIM + 1]
        cols.append(num / (den + 1e-16))
    o = jnp.concatenate(cols, axis=1) + b_ref[...]
    m = jnp.mean(o, axis=-1, keepdims=True)
    v = jnp.mean((o - m) ** 2, axis=-1, keepdims=True)
    o = (o - m) / jnp.sqrt(v + 1e-5) * g_ref[...] + lb_ref[...]
    o_ref[...] = jnp.maximum(o, 0.0)


def _finalize(acc, b, g, lb):
    N = acc.shape[1]
    BLK = 2048
    return pl.pallas_call(
        _fin_body,
        grid=(N // BLK,),
        in_specs=[
            pl.BlockSpec((HEADS, BLK, ACCW), lambda i: (0, i, 0)),
            pl.BlockSpec((HIDDEN,), lambda i: (0,)),
            pl.BlockSpec((HIDDEN,), lambda i: (0,)),
            pl.BlockSpec((HIDDEN,), lambda i: (0,)),
        ],
        out_specs=pl.BlockSpec((BLK, HIDDEN), lambda i: (i, 0)),
        out_shape=jax.ShapeDtypeStruct((N, HIDDEN), jnp.float32),
    )(acc, b, g, lb)


def _score_body(u_ref, i_ref, o_ref):
    o_ref[...] = jnp.sum(u_ref[...] * i_ref[...], axis=1)


def _scores(ue, ie):
    B = ue.shape[0]
    BLK = 2048
    return pl.pallas_call(
        _score_body,
        grid=(B // BLK,),
        in_specs=[
            pl.BlockSpec((BLK, HIDDEN), lambda i: (i, 0)),
            pl.BlockSpec((BLK, HIDDEN), lambda i: (i, 0)),
        ],
        out_specs=pl.BlockSpec((BLK,), lambda i: (i,)),
        out_shape=jax.ShapeDtypeStruct((B,), jnp.float32),
    )(ue, ie)


def _gather_body(uid_hbm, iid_hbm, ut_hbm, it_hbm, um_hbm, im_hbm,
                 xe_hbm, xm_hbm, ids_v, mm_v, rows_v, sem):
    c = lax.axis_index("c")
    s = lax.axis_index("s")
    wid = s * NC + c
    B = uid_hbm.shape[0]
    chunk = B // (NC * NS)
    nb = chunk // 128

    def one(idx_hbm, table_hbm, mmtab_hbm, out_base):
        base = wid * chunk
        pltpu.sync_copy(idx_hbm.at[pl.ds(base, chunk)], ids_v)
        descs = [pltpu.async_copy(table_hbm.at[ids_v.at[pl.ds(j * 128, 128)]],
                                  rows_v.at[pl.ds(j * 128, 128)], sem)
                 for j in range(nb)]
        def mmb(k, carry):
            v = ids_v[pl.ds(k * 16, 16)]
            mm_v[pl.ds(k * 16, 16)] = jnp.bitwise_and(v, 16383)
            return carry
        lax.fori_loop(0, chunk // 16, mmb, 0)
        for d in descs:
            d.wait()
        pltpu.sync_copy(rows_v, xe_hbm.at[pl.ds(out_base + base, chunk)])
        descs = [pltpu.async_copy(mmtab_hbm.at[mm_v.at[pl.ds(j * 128, 128)]],
                                  rows_v.at[pl.ds(j * 128, 128)], sem)
                 for j in range(nb)]
        for d in descs:
            d.wait()
        pltpu.sync_copy(rows_v, xm_hbm.at[pl.ds(out_base + base, chunk)])

    one(uid_hbm, ut_hbm, um_hbm, 0)
    one(iid_hbm, it_hbm, im_hbm, B)


def _sc_gather(uid, iid, user_table, item_table, UM, IM):
    B = uid.shape[0]
    N = 2 * B
    chunk = B // (NC * NS)
    k = pl.kernel(
        _gather_body,
        out_type=[
            jax.ShapeDtypeStruct((N, HIDDEN), jnp.float32),
            jax.ShapeDtypeStruct((N, HIDDEN), jnp.float32),
        ],
        mesh=_MESH,
        scratch_types=[
            pltpu.VMEM((chunk,), jnp.int32),
            pltpu.VMEM((chunk,), jnp.int32),
            pltpu.VMEM((chunk, HIDDEN), jnp.float32),
            pltpu.SemaphoreType.DMA,
        ],
        compiler_params=_SC_PARAMS,
    )
    return k(uid, iid, user_table, item_table, UM, IM)


def _w_body(src_hbm, dstr_hbm, ast_hbm, adt_hbm, w_hbm,
            asrc_v, adst_v, srcf_v, dst2_v, wblk_v):
    c = lax.axis_index("c")
    tid = lax.axis_index("s")
    E = src_hbm.shape[0]
    per_tile = E // NS
    ngroups = per_tile // GROUP
    zero16 = jnp.zeros((16,), jnp.int32)

    for r in range(2):
        head = r * 2 + c
        pltpu.sync_copy(ast_hbm.at[head], asrc_v)
        pltpu.sync_copy(adt_hbm.at[head], adst_v)

        def group_body(g, carry):
            base = tid * per_tile + g * GROUP
            pltpu.sync_copy(src_hbm.at[pl.ds(base, GROUP)], srcf_v)
            pltpu.sync_copy(dstr_hbm.at[tid * ngroups + g], dst2_v)

            def wbody(v, carry2):
                si = srcf_v[pl.ds(v * 16, 16)]
                di = dst2_v[v // 8, pl.ds((v % 8) * 16, 16)]
                e = (plsc.load_gather(asrc_v, [zero16, si])
                     + plsc.load_gather(adst_v, [zero16, di]))
                e = jnp.maximum(e, 0.2 * e)
                wblk_v[pl.ds(v * 16, 16)] = jnp.exp(e)
                return carry2
            lax.fori_loop(0, GROUP // 16, wbody, 0)
            pltpu.sync_copy(wblk_v, w_hbm.at[head].at[pl.ds(base, GROUP)])
            return carry
        lax.fori_loop(0, ngroups, group_body, 0)


def _sc_w(src, dst_r, ast, adt):
    N = ast.shape[2]
    E = src.shape[0]
    k = pl.kernel(
        _w_body,
        out_type=jax.ShapeDtypeStruct((HEADS, E), jnp.float32),
        mesh=_MESH,
        scratch_types=[
            pltpu.VMEM((1, N), jnp.float32),
            pltpu.VMEM((1, N), jnp.float32),
            pltpu.VMEM((GROUP,), jnp.int32),
            pltpu.VMEM((GROUP // 128, 128), jnp.int32),
            pltpu.VMEM((GROUP,), jnp.float32),
        ],
        compiler_params=_SC_PARAMS_UNTILED,
    )
    return k(src, dst_r, ast, adt)


def _edge_body(src_hbm, dstr_hbm, hh_hbm, w_hbm,
               out_hbm, acc_v, srcf_v, dst2_v, rows_v, msg_v, wblk_v, sem):
    c = lax.axis_index("c")
    tid = lax.axis_index("s")
    E = src_hbm.shape[0]
    N = acc_v.shape[0]
    per_tile = E // NS
    ngroups = per_tile // GROUP
    stripe = N // NS
    iota16 = lax.iota(jnp.int32, 16)
    zeros16 = jnp.zeros((16,), jnp.float32)

    def zero_msg():
        def zm(i, carry):
            msg_v[i, pl.ds(0, 16)] = zeros16
            msg_v[i, pl.ds(16, 16)] = zeros16
            msg_v[i, pl.ds(ACCW - 16, 16)] = zeros16
            return carry
        lax.fori_loop(0, GROUP, zm, 0)

    for r in range(2):
        head = r * 2 + c
        hh_head = hh_hbm.at[head]
        w_head = w_hbm.at[head]
        zero_msg()
        for z in range(stripe // GROUP):
            pltpu.sync_copy(
                msg_v, acc_v.at[pl.ds(tid * stripe + z * GROUP, GROUP)])
        plsc.subcore_barrier()

        def group_body(g, carry):
            base = tid * per_tile + g * GROUP
            pltpu.sync_copy(src_hbm.at[pl.ds(base, GROUP)], srcf_v)
            pltpu.sync_copy(dstr_hbm.at[tid * ngroups + g], dst2_v)
            pltpu.sync_copy(w_head.at[pl.ds(base, GROUP)], wblk_v)
            descs = [pltpu.async_copy(
                hh_head.at[srcf_v.at[pl.ds(j * 128, 128)]],
                rows_v.at[pl.ds(j * 128, 128)], sem)
                for j in range(GROUP // 128)]
            for d in descs:
                d.wait()

            def sbody(t, carry2):
                row_idx = t * 16 + iota16
                w16 = wblk_v[pl.ds(t * 16, 16)]
                for col in range(HEAD_DIM):
                    cidx = jnp.full((16,), col, jnp.int32)
                    colv = plsc.load_gather(rows_v, [row_idx, cidx])
                    plsc.store_scatter(msg_v, [row_idx, cidx], colv * w16)
                plsc.store_scatter(
                    msg_v, [row_idx, jnp.full((16,), HEAD_DIM, jnp.int32)],
                    w16)
                return carry2
            lax.fori_loop(0, GROUP // 16, sbody, 0)

            for j in range(GROUP // 128):
                pltpu.sync_copy(msg_v.at[pl.ds(j * 128, 128)],
                                acc_v.at[dst2_v.at[j]], add=True)
            return carry
        lax.fori_loop(0, ngroups, group_body, 0)
        plsc.subcore_barrier()
        pltpu.sync_copy(acc_v.at[pl.ds(tid * stripe, stripe)],
                        out_hbm.at[head].at[pl.ds(tid * stripe, stripe)])
        plsc.subcore_barrier()


def _sc_edge(src, dst_r, hh, w_all):
    N = hh.shape[1]
    k = pl.kernel(
        _edge_body,
        out_type=jax.ShapeDtypeStruct((HEADS, N, ACCW), jnp.float32),
        mesh=_MESH,
        scratch_types=[
            pltpu.VMEM_SHARED((N, ACCW), jnp.float32),
            pltpu.VMEM((GROUP,), jnp.int32),
            pltpu.VMEM((GROUP // 128, 128), jnp.int32),
            pltpu.VMEM((GROUP, HEAD_DIM), jnp.float32),
            pltpu.VMEM((GROUP, ACCW), jnp.float32),
            pltpu.VMEM((GROUP,), jnp.float32),
            pltpu.SemaphoreType.DMA,
        ],
        compiler_params=_SC_PARAMS_UNTILED,
    )
    return k(src, dst_r, hh, w_all)


def kernel(user_ids, item_ids, text_features, image_features, edge_index, params):
    p = params
    bsz = user_ids.shape[0]

    half = HIDDEN // 2
    M_u = p['ua']['Wv'] @ p['ua']['Wo']
    c_u = p['ua']['bv'] @ p['ua']['Wo'] + p['ua']['bo']
    M_i = p['ia']['Wv'] @ p['ia']['Wo']
    c_i = p['ia']['bv'] @ p['ia']['Wo'] + p['ia']['bo']
    W_t = jnp.concatenate([p['ut_W'] @ M_u[:half], p['it_W'] @ M_i[:half]], axis=1)
    W_im = jnp.concatenate([p['ui_W'] @ M_u[half:], p['ii_W'] @ M_i[half:]], axis=1)
    bias = jnp.concatenate([
        p['ut_b'] @ M_u[:half] + p['ui_b'] @ M_u[half:] + c_u,
        p['it_b'] @ M_i[:half] + p['ii_b'] @ M_i[half:] + c_i,
    ], axis=0)

    P = _feat_matmul(text_features, image_features, W_t, W_im, bias)
    UM_all = P[:, :HIDDEN]
    IM_all = P[:, HIDDEN:]

    x_emb, x_mm = _sc_gather(user_ids, item_ids, p['user_table'],
                             p['item_table'], UM_all, IM_all)

    src = edge_index[0]
    dst = edge_index[1]
    dst_r = dst.reshape(-1, GROUP // 128, 128)

    x = None
    for i in range(NUM_LAYERS):
        g = p['gat%d' % i]
        attm = jnp.concatenate([
            jax.scipy.linalg.block_diag(*[g['att_src'][k][:, None] for k in range(HEADS)]),
            jax.scipy.linalg.block_diag(*[g['att_dst'][k][:, None] for k in range(HEADS)]),
        ], axis=1)
        if i == 0:
            W = g['W']
            hh, a8 = _layer_pre0(x_emb, x_mm, W[:HIDDEN], W[HIDDEN:], attm)
        else:
            hh, a8 = _layer_pre(x, p['res%d_W' % (i - 1)], p['res%d_b' % (i - 1)],
                                g['W'], attm)
        a_t = a8.T
        w_all = _sc_w(src, dst_r,
                      a_t[:HEADS].reshape(HEADS, 1, -1),
                      a_t[HEADS:].reshape(HEADS, 1, -1))
        acc = _sc_edge(src, dst_r, hh, w_all)
        x = _finalize(acc, g['b'], p['ln%d_g' % i], p['ln%d_b' % i])

    ue = x[:bsz]
    ie = x[bsz:2 * bsz]
    return _scores(ue, ie), ue, ie


# speedup vs baseline: 31.7004x; 1.1378x over previous
import functools

import jax
import jax.numpy as jnp
from jax import lax
from jax.experimental import pallas as pl
from jax.experimental.pallas import tpu as pltpu
from jax.experimental.pallas import tpu_sc as plsc

HIDDEN = 128
HEADS = 4
HEAD_DIM = 32
NUM_LAYERS = 3
ACCW = 40
GROUP = 256
NC = 2
NS = 16

_MESH = plsc.VectorSubcoreMesh(core_axis_name="c", subcore_axis_name="s")
_SC_PARAMS = pltpu.CompilerParams(needs_layout_passes=False)
_SC_PARAMS_UNTILED = pltpu.CompilerParams(
    needs_layout_passes=False, use_tc_tiling_on_sc=False)


def _feat_body(t_ref, im_ref, wt_ref, wi_ref, b_ref, o_ref):
    acc = jnp.dot(t_ref[...], wt_ref[...], preferred_element_type=jnp.float32)
    acc += jnp.dot(im_ref[...], wi_ref[...], preferred_element_type=jnp.float32)
    o_ref[...] = acc + b_ref[...]


def _feat_matmul(text, image, W_t, W_i, bias):
    B = text.shape[0]
    BLK = 1024
    return pl.pallas_call(
        _feat_body,
        grid=(B // BLK,),
        in_specs=[
            pl.BlockSpec((BLK, text.shape[1]), lambda i: (i, 0)),
            pl.BlockSpec((BLK, image.shape[1]), lambda i: (i, 0)),
            pl.BlockSpec(W_t.shape, lambda i: (0, 0)),
            pl.BlockSpec(W_i.shape, lambda i: (0, 0)),
            pl.BlockSpec(bias.shape, lambda i: (0,)),
        ],
        out_specs=pl.BlockSpec((BLK, W_t.shape[1]), lambda i: (i, 0)),
        out_shape=jax.ShapeDtypeStruct((B, W_t.shape[1]), jnp.float32),
    )(text, image, W_t, W_i, bias)


def _pre0_body(xe_ref, xm_ref, w1_ref, w2_ref, att_ref, hh_ref, a8_ref):
    h = jnp.dot(xe_ref[...], w1_ref[...], preferred_element_type=jnp.float32)
    h += jnp.dot(xm_ref[...], w2_ref[...], preferred_element_type=jnp.float32)
    for k in range(HEADS):
        hh_ref[k] = h[:, k * HEAD_DIM:(k + 1) * HEAD_DIM]
    a8_ref[...] = jnp.dot(h, att_ref[...], preferred_element_type=jnp.float32)


def _layer_pre0(x_emb, x_mm, W1, W2, attm):
    N = x_emb.shape[0]
    BLK = 2048
    return pl.pallas_call(
        _pre0_body,
        grid=(N // BLK,),
        in_specs=[
            pl.BlockSpec((BLK, HIDDEN), lambda i: (i, 0)),
            pl.BlockSpec((BLK, HIDDEN), lambda i: (i, 0)),
            pl.BlockSpec((HIDDEN, HIDDEN), lambda i: (0, 0)),
            pl.BlockSpec((HIDDEN, HIDDEN), lambda i: (0, 0)),
            pl.BlockSpec((HIDDEN, 2 * HEADS), lambda i: (0, 0)),
        ],
        out_specs=[
            pl.BlockSpec((HEADS, BLK, HEAD_DIM), lambda i: (0, i, 0)),
            pl.BlockSpec((BLK, 2 * HEADS), lambda i: (i, 0)),
        ],
        out_shape=[
            jax.ShapeDtypeStruct((HEADS, N, HEAD_DIM), jnp.float32),
            jax.ShapeDtypeStruct((N, 2 * HEADS), jnp.float32),
        ],
    )(x_emb, x_mm, W1, W2, attm)


def _pre_body(x_ref, rw_ref, rb_ref, w_ref, att_ref, hh_ref, a8_ref):
    x = x_ref[...]
    xr = x + jnp.dot(x, rw_ref[...], preferred_element_type=jnp.float32) + rb_ref[...]
    h = jnp.dot(xr, w_ref[...], preferred_element_type=jnp.float32)
    for k in range(HEADS):
        hh_ref[k] = h[:, k * HEAD_DIM:(k + 1) * HEAD_DIM]
    a8_ref[...] = jnp.dot(h, att_ref[...], preferred_element_type=jnp.float32)


def _layer_pre(x, resW, resb, W, attm):
    N = x.shape[0]
    BLK = 2048
    return pl.pallas_call(
        _pre_body,
        grid=(N // BLK,),
        in_specs=[
            pl.BlockSpec((BLK, HIDDEN), lambda i: (i, 0)),
            pl.BlockSpec((HIDDEN, HIDDEN), lambda i: (0, 0)),
            pl.BlockSpec((HIDDEN,), lambda i: (0,)),
            pl.BlockSpec((HIDDEN, HIDDEN), lambda i: (0, 0)),
            pl.BlockSpec((HIDDEN, 2 * HEADS), lambda i: (0, 0)),
        ],
        out_specs=[
            pl.BlockSpec((HEADS, BLK, HEAD_DIM), lambda i: (0, i, 0)),
            pl.BlockSpec((BLK, 2 * HEADS), lambda i: (i, 0)),
        ],
        out_shape=[
            jax.ShapeDtypeStruct((HEADS, N, HEAD_DIM), jnp.float32),
            jax.ShapeDtypeStruct((N, 2 * HEADS), jnp.float32),
        ],
    )(x, resW, resb, W, attm)


def _fin_body(acc_ref, b_ref, g_ref, lb_ref, o_ref):
    cols = []
    for k in range(HEADS):
        a = acc_ref[k]
        num = a[:, :HEAD_DIM]
        den = a[:, HEAD_DIM:HEAD_DIM + 1]
        cols.append(num / (den + 1e-16))
    o = jnp.concatenate(cols, axis=1) + b_ref[...]
    m = jnp.mean(o, axis=-1, keepdims=True)
    v = jnp.mean((o - m) ** 2, axis=-1, keepdims=True)
    o = (o - m) / jnp.sqrt(v + 1e-5) * g_ref[...] + lb_ref[...]
    o_ref[...] = jnp.maximum(o, 0.0)


def _finalize(acc, b, g, lb):
    N = acc.shape[1]
    BLK = 2048
    return pl.pallas_call(
        _fin_body,
        grid=(N // BLK,),
        in_specs=[
            pl.BlockSpec((HEADS, BLK, ACCW), lambda i: (0, i, 0)),
            pl.BlockSpec((HIDDEN,), lambda i: (0,)),
            pl.BlockSpec((HIDDEN,), lambda i: (0,)),
            pl.BlockSpec((HIDDEN,), lambda i: (0,)),
        ],
        out_specs=pl.BlockSpec((BLK, HIDDEN), lambda i: (i, 0)),
        out_shape=jax.ShapeDtypeStruct((N, HIDDEN), jnp.float32),
    )(acc, b, g, lb)


def _score_body(u_ref, i_ref, o_ref):
    o_ref[...] = jnp.sum(u_ref[...] * i_ref[...], axis=1)


def _scores(ue, ie):
    B = ue.shape[0]
    BLK = 2048
    return pl.pallas_call(
        _score_body,
        grid=(B // BLK,),
        in_specs=[
            pl.BlockSpec((BLK, HIDDEN), lambda i: (i, 0)),
            pl.BlockSpec((BLK, HIDDEN), lambda i: (i, 0)),
        ],
        out_specs=pl.BlockSpec((BLK,), lambda i: (i,)),
        out_shape=jax.ShapeDtypeStruct((B,), jnp.float32),
    )(ue, ie)


def _gather_body(uid_hbm, iid_hbm, ut_hbm, it_hbm, um_hbm, im_hbm,
                 xe_hbm, xm_hbm, ids_v, mm_v, rows_v, sem):
    c = lax.axis_index("c")
    s = lax.axis_index("s")
    wid = s * NC + c
    B = uid_hbm.shape[0]
    chunk = B // (NC * NS)
    nb = chunk // 128

    def one(idx_hbm, table_hbm, mmtab_hbm, out_base):
        base = wid * chunk
        pltpu.sync_copy(idx_hbm.at[pl.ds(base, chunk)], ids_v)
        descs = [pltpu.async_copy(table_hbm.at[ids_v.at[pl.ds(j * 128, 128)]],
                                  rows_v.at[pl.ds(j * 128, 128)], sem)
                 for j in range(nb)]
        def mmb(k, carry):
            v = ids_v[pl.ds(k * 16, 16)]
            mm_v[pl.ds(k * 16, 16)] = jnp.bitwise_and(v, 16383)
            return carry
        lax.fori_loop(0, chunk // 16, mmb, 0)
        for d in descs:
            d.wait()
        pltpu.sync_copy(rows_v, xe_hbm.at[pl.ds(out_base + base, chunk)])
        descs = [pltpu.async_copy(mmtab_hbm.at[mm_v.at[pl.ds(j * 128, 128)]],
                                  rows_v.at[pl.ds(j * 128, 128)], sem)
                 for j in range(nb)]
        for d in descs:
            d.wait()
        pltpu.sync_copy(rows_v, xm_hbm.at[pl.ds(out_base + base, chunk)])

    one(uid_hbm, ut_hbm, um_hbm, 0)
    one(iid_hbm, it_hbm, im_hbm, B)


def _sc_gather(uid, iid, user_table, item_table, UM, IM):
    B = uid.shape[0]
    N = 2 * B
    chunk = B // (NC * NS)
    k = pl.kernel(
        _gather_body,
        out_type=[
            jax.ShapeDtypeStruct((N, HIDDEN), jnp.float32),
            jax.ShapeDtypeStruct((N, HIDDEN), jnp.float32),
        ],
        mesh=_MESH,
        scratch_types=[
            pltpu.VMEM((chunk,), jnp.int32),
            pltpu.VMEM((chunk,), jnp.int32),
            pltpu.VMEM((chunk, HIDDEN), jnp.float32),
            pltpu.SemaphoreType.DMA,
        ],
        compiler_params=_SC_PARAMS,
    )
    return k(uid, iid, user_table, item_table, UM, IM)


def _w_body(sd_hbm, ast_hbm, adt_hbm, w_hbm,
            asrc_v, adst_v, sd_v, wblk_v):
    c = lax.axis_index("c")
    tid = lax.axis_index("s")
    E = w_hbm.shape[1]
    per_tile = E // NS
    ngroups = per_tile // GROUP
    zero16 = jnp.zeros((16,), jnp.int32)

    for r in range(2):
        head = r * 2 + c
        pltpu.sync_copy(ast_hbm.at[head], asrc_v)
        pltpu.sync_copy(adt_hbm.at[head], adst_v)

        def group_body(g, carry):
            base = tid * per_tile + g * GROUP
            pltpu.sync_copy(sd_hbm.at[tid * ngroups + g], sd_v)

            def wbody(v, carry2):
                si = sd_v[v // 8, pl.ds((v % 8) * 16, 16)]
                di = sd_v[2 + v // 8, pl.ds((v % 8) * 16, 16)]
                e = (plsc.load_gather(asrc_v, [zero16, si])
                     + plsc.load_gather(adst_v, [zero16, di]))
                e = jnp.maximum(e, 0.2 * e)
                wblk_v[pl.ds(v * 16, 16)] = jnp.exp(e)
                return carry2
            lax.fori_loop(0, GROUP // 16, wbody, 0)
            pltpu.sync_copy(wblk_v, w_hbm.at[head].at[pl.ds(base, GROUP)])
            return carry
        lax.fori_loop(0, ngroups, group_body, 0)


def _sc_w(sd, ast, adt, E):
    N = ast.shape[2]
    k = pl.kernel(
        _w_body,
        out_type=jax.ShapeDtypeStruct((HEADS, E), jnp.float32),
        mesh=_MESH,
        scratch_types=[
            pltpu.VMEM((1, N), jnp.float32),
            pltpu.VMEM((1, N), jnp.float32),
            pltpu.VMEM((4, 128), jnp.int32),
            pltpu.VMEM((GROUP,), jnp.float32),
        ],
        compiler_params=_SC_PARAMS_UNTILED,
    )
    return k(sd, ast, adt)


def _edge_body(sd_hbm, hh_hbm, w_hbm, out_hbm, acc_v,
               sd0, sd1, sd2, sd3, wk0, wk1, wk2, wk3,
               rows0, rows1, msg0, msg1, sem_sdw, sem_g, sem_s):
    sdb = [sd0, sd1, sd2, sd3]
    wb = [wk0, wk1, wk2, wk3]
    rowsb = [rows0, rows1]
    msgb = [msg0, msg1]
    c = lax.axis_index("c")
    tid = lax.axis_index("s")
    E = w_hbm.shape[1]
    N = acc_v.shape[0]
    per_tile = E // NS
    ngroups = per_tile // GROUP
    stripe = N // NS
    iota16 = lax.iota(jnp.int32, 16)
    zeros16 = jnp.zeros((16,), jnp.float32)

    def zero_msg(m):
        def zm(i, carry):
            m[i, pl.ds(0, 16)] = zeros16
            m[i, pl.ds(16, 16)] = zeros16
            m[i, pl.ds(ACCW - 16, 16)] = zeros16
            return carry
        lax.fori_loop(0, GROUP, zm, 0)

    for r in range(2):
        head = r * 2 + c
        hh_head = hh_hbm.at[head]
        w_head = w_hbm.at[head]
        zero_msg(msg0)
        zero_msg(msg1)
        for z in range(stripe // GROUP):
            pltpu.sync_copy(
                msg0, acc_v.at[pl.ds(tid * stripe + z * GROUP, GROUP)])
        plsc.subcore_barrier()

        def fire_sdw(g, u):
            pltpu.async_copy(sd_hbm.at[tid * ngroups + g], sdb[u % 4],
                             sem_sdw)
            pltpu.async_copy(
                w_head.at[pl.ds(tid * per_tile + g * GROUP, GROUP)],
                wb[u % 4], sem_sdw)

        def wait_sdw(g, u):
            pltpu.make_async_copy(sd_hbm.at[tid * ngroups + g], sdb[u % 4],
                                  sem_sdw).wait()
            pltpu.make_async_copy(
                w_head.at[pl.ds(tid * per_tile + g * GROUP, GROUP)],
                wb[u % 4], sem_sdw).wait()

        def fire_g(g, u):
            for j in range(GROUP // 128):
                pltpu.async_copy(hh_head.at[sdb[u % 4].at[j]],
                                 rowsb[u % 2].at[pl.ds(j * 128, 128)], sem_g)

        def wait_g(g, u):
            for j in range(GROUP // 128):
                pltpu.make_async_copy(
                    hh_head.at[sdb[u % 4].at[j]],
                    rowsb[u % 2].at[pl.ds(j * 128, 128)], sem_g).wait()

        def fire_s(u):
            for j in range(GROUP // 128):
                pltpu.async_copy(msgb[u % 2].at[pl.ds(j * 128, 128)],
                                 acc_v.at[sdb[u % 4].at[2 + j]], sem_s,
                                 add=True)

        def wait_s(u):
            for j in range(GROUP // 128):
                pltpu.make_async_copy(msgb[u % 2].at[pl.ds(j * 128, 128)],
                                      acc_v.at[sdb[u % 4].at[2 + j]],
                                      sem_s).wait()

        def sbody(u):
            def inner(t, carry2):
                row_idx = t * 16 + iota16
                w16 = wb[u % 4][pl.ds(t * 16, 16)]
                for col in range(HEAD_DIM):
                    cidx = jnp.full((16,), col, jnp.int32)
                    colv = plsc.load_gather(rowsb[u % 2], [row_idx, cidx])
                    plsc.store_scatter(msgb[u % 2], [row_idx, cidx],
                                       colv * w16)
                plsc.store_scatter(
                    msgb[u % 2],
                    [row_idx, jnp.full((16,), HEAD_DIM, jnp.int32)], w16)
                return carry2
            lax.fori_loop(0, GROUP // 16, inner, 0)

        fire_sdw(0, 0)
        fire_sdw(1, 1)
        wait_sdw(0, 0)
        fire_g(0, 0)

        def outer(g0, carry):
            for u in range(4):
                g = g0 * 4 + u

                @pl.when(g <= ngroups - 2)
                def _():
                    wait_sdw(g + 1, u + 1)
                    fire_g(g + 1, u + 1)
                wait_g(g, u)
                sbody(u)

                @pl.when(g >= 1)
                def _():
                    wait_s(u - 1)
                fire_s(u)

                @pl.when(g <= ngroups - 3)
                def _():
                    fire_sdw(g + 2, u + 2)
            return carry
        lax.fori_loop(0, ngroups // 4, outer, 0)
        wait_s((ngroups - 1) % 4)
        plsc.subcore_barrier()
        pltpu.sync_copy(acc_v.at[pl.ds(tid * stripe, stripe)],
                        out_hbm.at[head].at[pl.ds(tid * stripe, stripe)])
        plsc.subcore_barrier()


def _sc_edge(sd, hh, w_all):
    N = hh.shape[1]
    k = pl.kernel(
        _edge_body,
        out_type=jax.ShapeDtypeStruct((HEADS, N, ACCW), jnp.float32),
        mesh=_MESH,
        scratch_types=(
            [pltpu.VMEM_SHARED((N, ACCW), jnp.float32)]
            + [pltpu.VMEM((4, 128), jnp.int32) for _ in range(4)]
            + [pltpu.VMEM((GROUP,), jnp.float32) for _ in range(4)]
            + [pltpu.VMEM((GROUP, HEAD_DIM), jnp.float32) for _ in range(2)]
            + [pltpu.VMEM((GROUP, ACCW), jnp.float32) for _ in range(2)]
            + [pltpu.SemaphoreType.DMA for _ in range(3)]
        ),
        compiler_params=_SC_PARAMS_UNTILED,
    )
    return k(sd, hh, w_all)


def kernel(user_ids, item_ids, text_features, image_features, edge_index, params):
    p = params
    bsz = user_ids.shape[0]

    half = HIDDEN // 2
    M_u = p['ua']['Wv'] @ p['ua']['Wo']
    c_u = p['ua']['bv'] @ p['ua']['Wo'] + p['ua']['bo']
    M_i = p['ia']['Wv'] @ p['ia']['Wo']
    c_i = p['ia']['bv'] @ p['ia']['Wo'] + p['ia']['bo']
    W_t = jnp.concatenate([p['ut_W'] @ M_u[:half], p['it_W'] @ M_i[:half]], axis=1)
    W_im = jnp.concatenate([p['ui_W'] @ M_u[half:], p['ii_W'] @ M_i[half:]], axis=1)
    bias = jnp.concatenate([
        p['ut_b'] @ M_u[:half] + p['ui_b'] @ M_u[half:] + c_u,
        p['it_b'] @ M_i[:half] + p['ii_b'] @ M_i[half:] + c_i,
    ], axis=0)

    P = _feat_matmul(text_features, image_features, W_t, W_im, bias)
    UM_all = P[:, :HIDDEN]
    IM_all = P[:, HIDDEN:]

    x_emb, x_mm = _sc_gather(user_ids, item_ids, p['user_table'],
                             p['item_table'], UM_all, IM_all)

    src = edge_index[0]
    dst = edge_index[1]
    E = src.shape[0]
    sd = jnp.concatenate([src.reshape(-1, 2, 128), dst.reshape(-1, 2, 128)],
                         axis=1)

    x = None
    for i in range(NUM_LAYERS):
        g = p['gat%d' % i]
        attm = jnp.concatenate([
            jax.scipy.linalg.block_diag(*[g['att_src'][k][:, None] for k in range(HEADS)]),
            jax.scipy.linalg.block_diag(*[g['att_dst'][k][:, None] for k in range(HEADS)]),
        ], axis=1)
        if i == 0:
            W = g['W']
            hh, a8 = _layer_pre0(x_emb, x_mm, W[:HIDDEN], W[HIDDEN:], attm)
        else:
            hh, a8 = _layer_pre(x, p['res%d_W' % (i - 1)], p['res%d_b' % (i - 1)],
                                g['W'], attm)
        a_t = a8.T
        w_all = _sc_w(sd, a_t[:HEADS].reshape(HEADS, 1, -1),
                      a_t[HEADS:].reshape(HEADS, 1, -1), E)
        acc = _sc_edge(sd, hh, w_all)
        x = _finalize(acc, g['b'], p['ln%d_g' % i], p['ln%d_b' % i])

    ue = x[:bsz]
    ie = x[bsz:2 * bsz]
    return _scores(ue, ie), ue, ie


# speedup vs baseline: 59.6849x; 1.8828x over previous
import functools

import jax
import jax.numpy as jnp
from jax import lax
from jax.experimental import pallas as pl
from jax.experimental.pallas import tpu as pltpu
from jax.experimental.pallas import tpu_sc as plsc

HIDDEN = 128
HEADS = 4
HEAD_DIM = 32
NUM_LAYERS = 3
ACCW = 40
GROUP = 256
NC = 2
NS = 16

_MESH = plsc.VectorSubcoreMesh(core_axis_name="c", subcore_axis_name="s")
_SC_PARAMS = pltpu.CompilerParams(needs_layout_passes=False)
_SC_PARAMS_UNTILED = pltpu.CompilerParams(
    needs_layout_passes=False, use_tc_tiling_on_sc=False)


def _feat_body(t_ref, im_ref, wt_ref, wi_ref, b_ref, o_ref):
    acc = jnp.dot(t_ref[...], wt_ref[...], preferred_element_type=jnp.float32)
    acc += jnp.dot(im_ref[...], wi_ref[...], preferred_element_type=jnp.float32)
    o_ref[...] = acc + b_ref[...]


def _feat_matmul(text, image, W_t, W_i, bias):
    B = text.shape[0]
    BLK = 1024
    return pl.pallas_call(
        _feat_body,
        grid=(B // BLK,),
        in_specs=[
            pl.BlockSpec((BLK, text.shape[1]), lambda i: (i, 0)),
            pl.BlockSpec((BLK, image.shape[1]), lambda i: (i, 0)),
            pl.BlockSpec(W_t.shape, lambda i: (0, 0)),
            pl.BlockSpec(W_i.shape, lambda i: (0, 0)),
            pl.BlockSpec(bias.shape, lambda i: (0,)),
        ],
        out_specs=pl.BlockSpec((BLK, W_t.shape[1]), lambda i: (i, 0)),
        out_shape=jax.ShapeDtypeStruct((B, W_t.shape[1]), jnp.float32),
    )(text, image, W_t, W_i, bias)


def _pre0_body(xe_ref, xm_ref, w1_ref, w2_ref, att_ref, hh_ref, a8_ref):
    h = jnp.dot(xe_ref[...], w1_ref[...], preferred_element_type=jnp.float32)
    h += jnp.dot(xm_ref[...], w2_ref[...], preferred_element_type=jnp.float32)
    for k in range(HEADS):
        hh_ref[k] = h[:, k * HEAD_DIM:(k + 1) * HEAD_DIM]
    a8_ref[...] = jnp.dot(h, att_ref[...], preferred_element_type=jnp.float32)


def _layer_pre0(x_emb, x_mm, W1, W2, attm):
    N = x_emb.shape[0]
    BLK = 2048
    return pl.pallas_call(
        _pre0_body,
        grid=(N // BLK,),
        in_specs=[
            pl.BlockSpec((BLK, HIDDEN), lambda i: (i, 0)),
            pl.BlockSpec((BLK, HIDDEN), lambda i: (i, 0)),
            pl.BlockSpec((HIDDEN, HIDDEN), lambda i: (0, 0)),
            pl.BlockSpec((HIDDEN, HIDDEN), lambda i: (0, 0)),
            pl.BlockSpec((HIDDEN, 2 * HEADS), lambda i: (0, 0)),
        ],
        out_specs=[
            pl.BlockSpec((HEADS, BLK, HEAD_DIM), lambda i: (0, i, 0)),
            pl.BlockSpec((BLK, 2 * HEADS), lambda i: (i, 0)),
        ],
        out_shape=[
            jax.ShapeDtypeStruct((HEADS, N, HEAD_DIM), jnp.float32),
            jax.ShapeDtypeStruct((N, 2 * HEADS), jnp.float32),
        ],
    )(x_emb, x_mm, W1, W2, attm)


def _pre_body(x_ref, rw_ref, rb_ref, w_ref, att_ref, hh_ref, a8_ref):
    x = x_ref[...]
    xr = x + jnp.dot(x, rw_ref[...], preferred_element_type=jnp.float32) + rb_ref[...]
    h = jnp.dot(xr, w_ref[...], preferred_element_type=jnp.float32)
    for k in range(HEADS):
        hh_ref[k] = h[:, k * HEAD_DIM:(k + 1) * HEAD_DIM]
    a8_ref[...] = jnp.dot(h, att_ref[...], preferred_element_type=jnp.float32)


def _layer_pre(x, resW, resb, W, attm):
    N = x.shape[0]
    BLK = 2048
    return pl.pallas_call(
        _pre_body,
        grid=(N // BLK,),
        in_specs=[
            pl.BlockSpec((BLK, HIDDEN), lambda i: (i, 0)),
            pl.BlockSpec((HIDDEN, HIDDEN), lambda i: (0, 0)),
            pl.BlockSpec((HIDDEN,), lambda i: (0,)),
            pl.BlockSpec((HIDDEN, HIDDEN), lambda i: (0, 0)),
            pl.BlockSpec((HIDDEN, 2 * HEADS), lambda i: (0, 0)),
        ],
        out_specs=[
            pl.BlockSpec((HEADS, BLK, HEAD_DIM), lambda i: (0, i, 0)),
            pl.BlockSpec((BLK, 2 * HEADS), lambda i: (i, 0)),
        ],
        out_shape=[
            jax.ShapeDtypeStruct((HEADS, N, HEAD_DIM), jnp.float32),
            jax.ShapeDtypeStruct((N, 2 * HEADS), jnp.float32),
        ],
    )(x, resW, resb, W, attm)


def _fin_body(acc_ref, b_ref, g_ref, lb_ref, o_ref):
    cols = []
    for k in range(HEADS):
        a = acc_ref[k]
        num = a[:, :HEAD_DIM]
        den = a[:, HEAD_DIM:HEAD_DIM + 1]
        cols.append(num / (den + 1e-16))
    o = jnp.concatenate(cols, axis=1) + b_ref[...]
    m = jnp.mean(o, axis=-1, keepdims=True)
    v = jnp.mean((o - m) ** 2, axis=-1, keepdims=True)
    o = (o - m) / jnp.sqrt(v + 1e-5) * g_ref[...] + lb_ref[...]
    o_ref[...] = jnp.maximum(o, 0.0)


def _finalize(acc, b, g, lb):
    N = acc.shape[1]
    BLK = 2048
    return pl.pallas_call(
        _fin_body,
        grid=(N // BLK,),
        in_specs=[
            pl.BlockSpec((HEADS, BLK, ACCW), lambda i: (0, i, 0)),
            pl.BlockSpec((HIDDEN,), lambda i: (0,)),
            pl.BlockSpec((HIDDEN,), lambda i: (0,)),
            pl.BlockSpec((HIDDEN,), lambda i: (0,)),
        ],
        out_specs=pl.BlockSpec((BLK, HIDDEN), lambda i: (i, 0)),
        out_shape=jax.ShapeDtypeStruct((N, HIDDEN), jnp.float32),
    )(acc, b, g, lb)


def _score_body(u_ref, i_ref, o_ref):
    o_ref[...] = jnp.sum(u_ref[...] * i_ref[...], axis=1)


def _scores(ue, ie):
    B = ue.shape[0]
    BLK = 2048
    return pl.pallas_call(
        _score_body,
        grid=(B // BLK,),
        in_specs=[
            pl.BlockSpec((BLK, HIDDEN), lambda i: (i, 0)),
            pl.BlockSpec((BLK, HIDDEN), lambda i: (i, 0)),
        ],
        out_specs=pl.BlockSpec((BLK,), lambda i: (i,)),
        out_shape=jax.ShapeDtypeStruct((B,), jnp.float32),
    )(ue, ie)


def _gather_body(uid_hbm, iid_hbm, ut_hbm, it_hbm, um_hbm, im_hbm,
                 xe_hbm, xm_hbm, ids_v, mm_v, rows_v, sem):
    c = lax.axis_index("c")
    s = lax.axis_index("s")
    wid = s * NC + c
    B = uid_hbm.shape[0]
    chunk = B // (NC * NS)
    nb = chunk // 128

    def one(idx_hbm, table_hbm, mmtab_hbm, out_base):
        base = wid * chunk
        pltpu.sync_copy(idx_hbm.at[pl.ds(base, chunk)], ids_v)
        descs = [pltpu.async_copy(table_hbm.at[ids_v.at[pl.ds(j * 128, 128)]],
                                  rows_v.at[pl.ds(j * 128, 128)], sem)
                 for j in range(nb)]
        def mmb(k, carry):
            v = ids_v[pl.ds(k * 16, 16)]
            mm_v[pl.ds(k * 16, 16)] = jnp.bitwise_and(v, 16383)
            return carry
        lax.fori_loop(0, chunk // 16, mmb, 0)
        for d in descs:
            d.wait()
        pltpu.sync_copy(rows_v, xe_hbm.at[pl.ds(out_base + base, chunk)])
        descs = [pltpu.async_copy(mmtab_hbm.at[mm_v.at[pl.ds(j * 128, 128)]],
                                  rows_v.at[pl.ds(j * 128, 128)], sem)
                 for j in range(nb)]
        for d in descs:
            d.wait()
        pltpu.sync_copy(rows_v, xm_hbm.at[pl.ds(out_base + base, chunk)])

    one(uid_hbm, ut_hbm, um_hbm, 0)
    one(iid_hbm, it_hbm, im_hbm, B)


def _sc_gather(uid, iid, user_table, item_table, UM, IM):
    B = uid.shape[0]
    N = 2 * B
    chunk = B // (NC * NS)
    k = pl.kernel(
        _gather_body,
        out_type=[
            jax.ShapeDtypeStruct((N, HIDDEN), jnp.float32),
            jax.ShapeDtypeStruct((N, HIDDEN), jnp.float32),
        ],
        mesh=_MESH,
        scratch_types=[
            pltpu.VMEM((chunk,), jnp.int32),
            pltpu.VMEM((chunk,), jnp.int32),
            pltpu.VMEM((chunk, HIDDEN), jnp.float32),
            pltpu.SemaphoreType.DMA,
        ],
        compiler_params=_SC_PARAMS,
    )
    return k(uid, iid, user_table, item_table, UM, IM)


def _w_body(sd_hbm, ast_hbm, adt_hbm, w_hbm,
            asrc_v, adst_v, sd_v, wblk_v):
    c = lax.axis_index("c")
    tid = lax.axis_index("s")
    E = w_hbm.shape[1]
    per_tile = E // NS
    ngroups = per_tile // GROUP
    zero16 = jnp.zeros((16,), jnp.int32)

    for r in range(2):
        head = r * 2 + c
        pltpu.sync_copy(ast_hbm.at[head], asrc_v)
        pltpu.sync_copy(adt_hbm.at[head], adst_v)

        def group_body(g, carry):
            base = tid * per_tile + g * GROUP
            pltpu.sync_copy(sd_hbm.at[tid * ngroups + g], sd_v)

            def wbody(v, carry2):
                si = sd_v[v // 8, pl.ds((v % 8) * 16, 16)]
                di = sd_v[2 + v // 8, pl.ds((v % 8) * 16, 16)]
                e = (plsc.load_gather(asrc_v, [zero16, si])
                     + plsc.load_gather(adst_v, [zero16, di]))
                e = jnp.maximum(e, 0.2 * e)
                wblk_v[pl.ds(v * 16, 16)] = jnp.exp(e)
                return carry2
            lax.fori_loop(0, GROUP // 16, wbody, 0)
            pltpu.sync_copy(wblk_v, w_hbm.at[head].at[pl.ds(base, GROUP)])
            return carry
        lax.fori_loop(0, ngroups, group_body, 0)


def _sc_w(sd, ast, adt, E):
    N = ast.shape[2]
    k = pl.kernel(
        _w_body,
        out_type=jax.ShapeDtypeStruct((HEADS, E), jnp.float32),
        mesh=_MESH,
        scratch_types=[
            pltpu.VMEM((1, N), jnp.float32),
            pltpu.VMEM((1, N), jnp.float32),
            pltpu.VMEM((4, 128), jnp.int32),
            pltpu.VMEM((GROUP,), jnp.float32),
        ],
        compiler_params=_SC_PARAMS_UNTILED,
    )
    return k(sd, ast, adt)


def _edge_body(sd_hbm, hh_hbm, w_hbm, out_hbm, acc_v,
               sd0, sd1, sd2, sd3, wk0, wk1, wk2, wk3,
               rows0, rows1, msg0, msg1, sem_sdw, sem_g, sem_s):
    sdb = [sd0, sd1, sd2, sd3]
    wb = [wk0, wk1, wk2, wk3]
    rowsb = [rows0, rows1]
    msgb = [msg0, msg1]
    c = lax.axis_index("c")
    tid = lax.axis_index("s")
    E = w_hbm.shape[1]
    N = acc_v.shape[0]
    per_tile = E // NS
    ngroups = per_tile // GROUP
    stripe = N // NS
    iota16 = lax.iota(jnp.int32, 16)
    zeros16 = jnp.zeros((16,), jnp.float32)

    def zero_msg(m):
        def zm(i, carry):
            m[i, pl.ds(0, 16)] = zeros16
            m[i, pl.ds(16, 16)] = zeros16
            m[i, pl.ds(ACCW - 16, 16)] = zeros16
            return carry
        lax.fori_loop(0, GROUP, zm, 0)

    for r in range(2):
        head = r * 2 + c
        hh_head = hh_hbm.at[head]
        w_head = w_hbm.at[head]
        zero_msg(msg0)
        zero_msg(msg1)
        for z in range(stripe // GROUP):
            pltpu.sync_copy(
                msg0, acc_v.at[pl.ds(tid * stripe + z * GROUP, GROUP)])
        plsc.subcore_barrier()

        def fire_sdw(g, u):
            pltpu.async_copy(sd_hbm.at[tid * ngroups + g], sdb[u % 4],
                             sem_sdw)
            pltpu.async_copy(
                w_head.at[pl.ds(tid * per_tile + g * GROUP, GROUP)],
                wb[u % 4], sem_sdw)

        def wait_sdw(g, u):
            pltpu.make_async_copy(sd_hbm.at[tid * ngroups + g], sdb[u % 4],
                                  sem_sdw).wait()
            pltpu.make_async_copy(
                w_head.at[pl.ds(tid * per_tile + g * GROUP, GROUP)],
                wb[u % 4], sem_sdw).wait()

        def fire_g(g, u):
            for j in range(GROUP // 128):
                pltpu.async_copy(hh_head.at[sdb[u % 4].at[j]],
                                 rowsb[u % 2].at[pl.ds(j * 128, 128)], sem_g)

        def wait_g(g, u):
            for j in range(GROUP // 128):
                pltpu.make_async_copy(
                    hh_head.at[sdb[u % 4].at[j]],
                    rowsb[u % 2].at[pl.ds(j * 128, 128)], sem_g).wait()

        def fire_s(u):
            for j in range(GROUP // 128):
                pltpu.async_copy(msgb[u % 2].at[pl.ds(j * 128, 128)],
                                 acc_v.at[sdb[u % 4].at[2 + j]], sem_s,
                                 add=True)

        def wait_s(u):
            for j in range(GROUP // 128):
                pltpu.make_async_copy(msgb[u % 2].at[pl.ds(j * 128, 128)],
                                      acc_v.at[sdb[u % 4].at[2 + j]],
                                      sem_s).wait()

        def sbody(u):
            def inner(t, carry2):
                base = t * 16
                w16 = wb[u % 4][pl.ds(base, 16)]
                for i in range(16):
                    r = base + i
                    wv = w16[i]
                    msgb[u % 2][r, pl.ds(0, 16)] = (
                        rowsb[u % 2][r, pl.ds(0, 16)] * wv)
                    msgb[u % 2][r, pl.ds(16, 16)] = (
                        rowsb[u % 2][r, pl.ds(16, 16)] * wv)
                plsc.store_scatter(
                    msgb[u % 2],
                    [base + iota16, jnp.full((16,), HEAD_DIM, jnp.int32)],
                    w16)
                return carry2
            lax.fori_loop(0, GROUP // 16, inner, 0)

        fire_sdw(0, 0)
        fire_sdw(1, 1)
        wait_sdw(0, 0)
        fire_g(0, 0)

        def outer(g0, carry):
            for u in range(4):
                g = g0 * 4 + u

                @pl.when(g <= ngroups - 2)
                def _():
                    wait_sdw(g + 1, u + 1)
                    fire_g(g + 1, u + 1)
                wait_g(g, u)
                sbody(u)

                @pl.when(g >= 1)
                def _():
                    wait_s(u - 1)
                fire_s(u)

                @pl.when(g <= ngroups - 3)
                def _():
                    fire_sdw(g + 2, u + 2)
            return carry
        lax.fori_loop(0, ngroups // 4, outer, 0)
        wait_s((ngroups - 1) % 4)
        plsc.subcore_barrier()
        pltpu.sync_copy(acc_v.at[pl.ds(tid * stripe, stripe)],
                        out_hbm.at[head].at[pl.ds(tid * stripe, stripe)])
        plsc.subcore_barrier()


def _sc_edge(sd, hh, w_all):
    N = hh.shape[1]
    k = pl.kernel(
        _edge_body,
        out_type=jax.ShapeDtypeStruct((HEADS, N, ACCW), jnp.float32),
        mesh=_MESH,
        scratch_types=(
            [pltpu.VMEM_SHARED((N, ACCW), jnp.float32)]
            + [pltpu.VMEM((4, 128), jnp.int32) for _ in range(4)]
            + [pltpu.VMEM((GROUP,), jnp.float32) for _ in range(4)]
            + [pltpu.VMEM((GROUP, HEAD_DIM), jnp.float32) for _ in range(2)]
            + [pltpu.VMEM((GROUP, ACCW), jnp.float32) for _ in range(2)]
            + [pltpu.SemaphoreType.DMA for _ in range(3)]
        ),
        compiler_params=_SC_PARAMS_UNTILED,
    )
    return k(sd, hh, w_all)


def kernel(user_ids, item_ids, text_features, image_features, edge_index, params):
    p = params
    bsz = user_ids.shape[0]

    half = HIDDEN // 2
    M_u = p['ua']['Wv'] @ p['ua']['Wo']
    c_u = p['ua']['bv'] @ p['ua']['Wo'] + p['ua']['bo']
    M_i = p['ia']['Wv'] @ p['ia']['Wo']
    c_i = p['ia']['bv'] @ p['ia']['Wo'] + p['ia']['bo']
    W_t = jnp.concatenate([p['ut_W'] @ M_u[:half], p['it_W'] @ M_i[:half]], axis=1)
    W_im = jnp.concatenate([p['ui_W'] @ M_u[half:], p['ii_W'] @ M_i[half:]], axis=1)
    bias = jnp.concatenate([
        p['ut_b'] @ M_u[:half] + p['ui_b'] @ M_u[half:] + c_u,
        p['it_b'] @ M_i[:half] + p['ii_b'] @ M_i[half:] + c_i,
    ], axis=0)

    P = _feat_matmul(text_features, image_features, W_t, W_im, bias)
    UM_all = P[:, :HIDDEN]
    IM_all = P[:, HIDDEN:]

    x_emb, x_mm = _sc_gather(user_ids, item_ids, p['user_table'],
                             p['item_table'], UM_all, IM_all)

    src = edge_index[0]
    dst = edge_index[1]
    E = src.shape[0]
    sd = jnp.concatenate([src.reshape(-1, 2, 128), dst.reshape(-1, 2, 128)],
                         axis=1)

    x = None
    for i in range(NUM_LAYERS):
        g = p['gat%d' % i]
        attm = jnp.concatenate([
            jax.scipy.linalg.block_diag(*[g['att_src'][k][:, None] for k in range(HEADS)]),
            jax.scipy.linalg.block_diag(*[g['att_dst'][k][:, None] for k in range(HEADS)]),
        ], axis=1)
        if i == 0:
            W = g['W']
            hh, a8 = _layer_pre0(x_emb, x_mm, W[:HIDDEN], W[HIDDEN:], attm)
        else:
            hh, a8 = _layer_pre(x, p['res%d_W' % (i - 1)], p['res%d_b' % (i - 1)],
                                g['W'], attm)
        a_t = a8.T
        w_all = _sc_w(sd, a_t[:HEADS].reshape(HEADS, 1, -1),
                      a_t[HEADS:].reshape(HEADS, 1, -1), E)
        acc = _sc_edge(sd, hh, w_all)
        x = _finalize(acc, g['b'], p['ln%d_g' % i], p['ln%d_b' % i])

    ue = x[:bsz]
    ie = x[bsz:2 * bsz]
    return _scores(ue, ie), ue, ie


# speedup vs baseline: 66.0591x; 1.1068x over previous
import functools

import jax
import jax.numpy as jnp
from jax import lax
from jax.experimental import pallas as pl
from jax.experimental.pallas import tpu as pltpu
from jax.experimental.pallas import tpu_sc as plsc

HIDDEN = 128
HEADS = 4
HEAD_DIM = 32
NUM_LAYERS = 3
ACCW = 40
GROUP = 256
NC = 2
NS = 16

_MESH = plsc.VectorSubcoreMesh(core_axis_name="c", subcore_axis_name="s")
_SC_PARAMS = pltpu.CompilerParams(needs_layout_passes=False)
_SC_PARAMS_UNTILED = pltpu.CompilerParams(
    needs_layout_passes=False, use_tc_tiling_on_sc=False)


def _feat_body(t_ref, im_ref, wt_ref, wi_ref, b_ref, o_ref):
    acc = jnp.dot(t_ref[...], wt_ref[...], preferred_element_type=jnp.float32)
    acc += jnp.dot(im_ref[...], wi_ref[...], preferred_element_type=jnp.float32)
    o_ref[...] = acc + b_ref[...]


def _feat_matmul(text, image, W_t, W_i, bias):
    B = text.shape[0]
    BLK = 1024
    return pl.pallas_call(
        _feat_body,
        grid=(B // BLK,),
        in_specs=[
            pl.BlockSpec((BLK, text.shape[1]), lambda i: (i, 0)),
            pl.BlockSpec((BLK, image.shape[1]), lambda i: (i, 0)),
            pl.BlockSpec(W_t.shape, lambda i: (0, 0)),
            pl.BlockSpec(W_i.shape, lambda i: (0, 0)),
            pl.BlockSpec(bias.shape, lambda i: (0,)),
        ],
        out_specs=pl.BlockSpec((BLK, W_t.shape[1]), lambda i: (i, 0)),
        out_shape=jax.ShapeDtypeStruct((B, W_t.shape[1]), jnp.float32),
    )(text, image, W_t, W_i, bias)


def _pre0_body(xe_ref, xm_ref, w1_ref, w2_ref, att_ref, hh_ref, a8_ref):
    h = jnp.dot(xe_ref[...], w1_ref[...], preferred_element_type=jnp.float32)
    h += jnp.dot(xm_ref[...], w2_ref[...], preferred_element_type=jnp.float32)
    for k in range(HEADS):
        hh_ref[k] = h[:, k * HEAD_DIM:(k + 1) * HEAD_DIM]
    a8_ref[...] = jnp.dot(h, att_ref[...], preferred_element_type=jnp.float32)


def _layer_pre0(x_emb, x_mm, W1, W2, attm):
    N = x_emb.shape[0]
    BLK = 2048
    return pl.pallas_call(
        _pre0_body,
        grid=(N // BLK,),
        in_specs=[
            pl.BlockSpec((BLK, HIDDEN), lambda i: (i, 0)),
            pl.BlockSpec((BLK, HIDDEN), lambda i: (i, 0)),
            pl.BlockSpec((HIDDEN, HIDDEN), lambda i: (0, 0)),
            pl.BlockSpec((HIDDEN, HIDDEN), lambda i: (0, 0)),
            pl.BlockSpec((HIDDEN, 2 * HEADS), lambda i: (0, 0)),
        ],
        out_specs=[
            pl.BlockSpec((HEADS, BLK, HEAD_DIM), lambda i: (0, i, 0)),
            pl.BlockSpec((BLK, 2 * HEADS), lambda i: (i, 0)),
        ],
        out_shape=[
            jax.ShapeDtypeStruct((HEADS, N, HEAD_DIM), jnp.float32),
            jax.ShapeDtypeStruct((N, 2 * HEADS), jnp.float32),
        ],
    )(x_emb, x_mm, W1, W2, attm)


def _pre_body(x_ref, rw_ref, rb_ref, w_ref, att_ref, hh_ref, a8_ref):
    x = x_ref[...]
    xr = x + jnp.dot(x, rw_ref[...], preferred_element_type=jnp.float32) + rb_ref[...]
    h = jnp.dot(xr, w_ref[...], preferred_element_type=jnp.float32)
    for k in range(HEADS):
        hh_ref[k] = h[:, k * HEAD_DIM:(k + 1) * HEAD_DIM]
    a8_ref[...] = jnp.dot(h, att_ref[...], preferred_element_type=jnp.float32)


def _layer_pre(x, resW, resb, W, attm):
    N = x.shape[0]
    BLK = 2048
    return pl.pallas_call(
        _pre_body,
        grid=(N // BLK,),
        in_specs=[
            pl.BlockSpec((BLK, HIDDEN), lambda i: (i, 0)),
            pl.BlockSpec((HIDDEN, HIDDEN), lambda i: (0, 0)),
            pl.BlockSpec((HIDDEN,), lambda i: (0,)),
            pl.BlockSpec((HIDDEN, HIDDEN), lambda i: (0, 0)),
            pl.BlockSpec((HIDDEN, 2 * HEADS), lambda i: (0, 0)),
        ],
        out_specs=[
            pl.BlockSpec((HEADS, BLK, HEAD_DIM), lambda i: (0, i, 0)),
            pl.BlockSpec((BLK, 2 * HEADS), lambda i: (i, 0)),
        ],
        out_shape=[
            jax.ShapeDtypeStruct((HEADS, N, HEAD_DIM), jnp.float32),
            jax.ShapeDtypeStruct((N, 2 * HEADS), jnp.float32),
        ],
    )(x, resW, resb, W, attm)


def _fin_body(acc_ref, b_ref, g_ref, lb_ref, o_ref):
    cols = []
    for k in range(HEADS):
        a = acc_ref[k]
        num = a[:, :HEAD_DIM]
        den = a[:, HEAD_DIM:HEAD_DIM + 1]
        cols.append(num / (den + 1e-16))
    o = jnp.concatenate(cols, axis=1) + b_ref[...]
    m = jnp.mean(o, axis=-1, keepdims=True)
    v = jnp.mean((o - m) ** 2, axis=-1, keepdims=True)
    o = (o - m) / jnp.sqrt(v + 1e-5) * g_ref[...] + lb_ref[...]
    o_ref[...] = jnp.maximum(o, 0.0)


def _finalize(acc, b, g, lb):
    N = acc.shape[1]
    BLK = 2048
    return pl.pallas_call(
        _fin_body,
        grid=(N // BLK,),
        in_specs=[
            pl.BlockSpec((HEADS, BLK, ACCW), lambda i: (0, i, 0)),
            pl.BlockSpec((HIDDEN,), lambda i: (0,)),
            pl.BlockSpec((HIDDEN,), lambda i: (0,)),
            pl.BlockSpec((HIDDEN,), lambda i: (0,)),
        ],
        out_specs=pl.BlockSpec((BLK, HIDDEN), lambda i: (i, 0)),
        out_shape=jax.ShapeDtypeStruct((N, HIDDEN), jnp.float32),
    )(acc, b, g, lb)


def _score_body(u_ref, i_ref, o_ref):
    o_ref[...] = jnp.sum(u_ref[...] * i_ref[...], axis=1)


def _scores(ue, ie):
    B = ue.shape[0]
    BLK = 2048
    return pl.pallas_call(
        _score_body,
        grid=(B // BLK,),
        in_specs=[
            pl.BlockSpec((BLK, HIDDEN), lambda i: (i, 0)),
            pl.BlockSpec((BLK, HIDDEN), lambda i: (i, 0)),
        ],
        out_specs=pl.BlockSpec((BLK,), lambda i: (i,)),
        out_shape=jax.ShapeDtypeStruct((B,), jnp.float32),
    )(ue, ie)


def _gather_body(uid_hbm, iid_hbm, ut_hbm, it_hbm, um_hbm, im_hbm,
                 xe_hbm, xm_hbm, ids_v, mm_v, rows_v, sem):
    c = lax.axis_index("c")
    s = lax.axis_index("s")
    wid = s * NC + c
    B = uid_hbm.shape[0]
    chunk = B // (NC * NS)
    nb = chunk // 128

    def one(idx_hbm, table_hbm, mmtab_hbm, out_base):
        base = wid * chunk
        pltpu.sync_copy(idx_hbm.at[pl.ds(base, chunk)], ids_v)
        descs = [pltpu.async_copy(table_hbm.at[ids_v.at[pl.ds(j * 128, 128)]],
                                  rows_v.at[pl.ds(j * 128, 128)], sem)
                 for j in range(nb)]
        def mmb(k, carry):
            v = ids_v[pl.ds(k * 16, 16)]
            mm_v[pl.ds(k * 16, 16)] = jnp.bitwise_and(v, 16383)
            return carry
        lax.fori_loop(0, chunk // 16, mmb, 0)
        for d in descs:
            d.wait()
        pltpu.sync_copy(rows_v, xe_hbm.at[pl.ds(out_base + base, chunk)])
        descs = [pltpu.async_copy(mmtab_hbm.at[mm_v.at[pl.ds(j * 128, 128)]],
                                  rows_v.at[pl.ds(j * 128, 128)], sem)
                 for j in range(nb)]
        for d in descs:
            d.wait()
        pltpu.sync_copy(rows_v, xm_hbm.at[pl.ds(out_base + base, chunk)])

    one(uid_hbm, ut_hbm, um_hbm, 0)
    one(iid_hbm, it_hbm, im_hbm, B)


def _sc_gather(uid, iid, user_table, item_table, UM, IM):
    B = uid.shape[0]
    N = 2 * B
    chunk = B // (NC * NS)
    k = pl.kernel(
        _gather_body,
        out_type=[
            jax.ShapeDtypeStruct((N, HIDDEN), jnp.float32),
            jax.ShapeDtypeStruct((N, HIDDEN), jnp.float32),
        ],
        mesh=_MESH,
        scratch_types=[
            pltpu.VMEM((chunk,), jnp.int32),
            pltpu.VMEM((chunk,), jnp.int32),
            pltpu.VMEM((chunk, HIDDEN), jnp.float32),
            pltpu.SemaphoreType.DMA,
        ],
        compiler_params=_SC_PARAMS,
    )
    return k(uid, iid, user_table, item_table, UM, IM)


def _w_body(sd_hbm, ast_hbm, adt_hbm, w_hbm,
            asrc_v, adst_v, sdv0, sdv1, wv0, wv1, sem_sd, sem_w):
    sdb = [sdv0, sdv1]
    wbk = [wv0, wv1]
    c = lax.axis_index("c")
    tid = lax.axis_index("s")
    E = w_hbm.shape[1]
    per_tile = E // NS
    ngroups = per_tile // GROUP
    zero16 = jnp.zeros((16,), jnp.int32)

    for r in range(2):
        head = r * 2 + c
        w_head = w_hbm.at[head]
        pltpu.sync_copy(ast_hbm.at[head], asrc_v)
        pltpu.sync_copy(adt_hbm.at[head], adst_v)

        def fire_sd(g, u):
            pltpu.async_copy(sd_hbm.at[tid * ngroups + g], sdb[u % 2],
                             sem_sd)

        def wait_sd(g, u):
            pltpu.make_async_copy(sd_hbm.at[tid * ngroups + g], sdb[u % 2],
                                  sem_sd).wait()

        def fire_w(g, u):
            pltpu.async_copy(
                wbk[u % 2],
                w_head.at[pl.ds(tid * per_tile + g * GROUP, GROUP)], sem_w)

        def wait_w(g, u):
            pltpu.make_async_copy(
                wbk[u % 2],
                w_head.at[pl.ds(tid * per_tile + g * GROUP, GROUP)],
                sem_w).wait()

        fire_sd(0, 0)
        fire_sd(1, 1)

        def outer(g0, carry):
            for u in range(2):
                g = g0 * 2 + u
                wait_sd(g, u)

                @pl.when(g >= 2)
                def _():
                    wait_w(g - 2, u)

                def wbody(v, carry2):
                    si = sdb[u % 2][v // 8, pl.ds((v % 8) * 16, 16)]
                    di = sdb[u % 2][2 + v // 8, pl.ds((v % 8) * 16, 16)]
                    e = (plsc.load_gather(asrc_v, [zero16, si])
                         + plsc.load_gather(adst_v, [zero16, di]))
                    e = jnp.maximum(e, 0.2 * e)
                    wbk[u % 2][pl.ds(v * 16, 16)] = jnp.exp(e)
                    return carry2
                lax.fori_loop(0, GROUP // 16, wbody, 0)
                fire_w(g, u)

                @pl.when(g + 2 <= ngroups - 1)
                def _():
                    fire_sd(g + 2, u)
            return carry
        lax.fori_loop(0, ngroups // 2, outer, 0)
        wait_w(ngroups - 2, 0)
        wait_w(ngroups - 1, 1)


def _sc_w(sd, ast, adt, E):
    N = ast.shape[2]
    k = pl.kernel(
        _w_body,
        out_type=jax.ShapeDtypeStruct((HEADS, E), jnp.float32),
        mesh=_MESH,
        scratch_types=[
            pltpu.VMEM((1, N), jnp.float32),
            pltpu.VMEM((1, N), jnp.float32),
            pltpu.VMEM((4, 128), jnp.int32),
            pltpu.VMEM((4, 128), jnp.int32),
            pltpu.VMEM((GROUP,), jnp.float32),
            pltpu.VMEM((GROUP,), jnp.float32),
            pltpu.SemaphoreType.DMA,
            pltpu.SemaphoreType.DMA,
        ],
        compiler_params=_SC_PARAMS_UNTILED,
    )
    return k(sd, ast, adt)


def _edge_body(sd_hbm, hh_hbm, w_hbm, out_hbm, acc_v,
               sd0, sd1, sd2, sd3, wk0, wk1, wk2, wk3,
               rows0, rows1, msg0, msg1, sem_sdw, sem_g, sem_s):
    sdb = [sd0, sd1, sd2, sd3]
    wb = [wk0, wk1, wk2, wk3]
    rowsb = [rows0, rows1]
    msgb = [msg0, msg1]
    c = lax.axis_index("c")
    tid = lax.axis_index("s")
    E = w_hbm.shape[1]
    N = acc_v.shape[0]
    per_tile = E // NS
    ngroups = per_tile // GROUP
    stripe = N // NS
    iota16 = lax.iota(jnp.int32, 16)
    zeros16 = jnp.zeros((16,), jnp.float32)

    def zero_msg(m):
        def zm(i, carry):
            m[i, pl.ds(0, 16)] = zeros16
            m[i, pl.ds(16, 16)] = zeros16
            m[i, pl.ds(ACCW - 16, 16)] = zeros16
            return carry
        lax.fori_loop(0, GROUP, zm, 0)

    for r in range(2):
        head = r * 2 + c
        hh_head = hh_hbm.at[head]
        w_head = w_hbm.at[head]
        zero_msg(msg0)
        zero_msg(msg1)
        for z in range(stripe // GROUP):
            pltpu.sync_copy(
                msg0, acc_v.at[pl.ds(tid * stripe + z * GROUP, GROUP)])
        plsc.subcore_barrier()

        def fire_sdw(g, u):
            pltpu.async_copy(sd_hbm.at[tid * ngroups + g], sdb[u % 4],
                             sem_sdw)
            pltpu.async_copy(
                w_head.at[pl.ds(tid * per_tile + g * GROUP, GROUP)],
                wb[u % 4], sem_sdw)

        def wait_sdw(g, u):
            pltpu.make_async_copy(sd_hbm.at[tid * ngroups + g], sdb[u % 4],
                                  sem_sdw).wait()
            pltpu.make_async_copy(
                w_head.at[pl.ds(tid * per_tile + g * GROUP, GROUP)],
                wb[u % 4], sem_sdw).wait()

        def fire_g(g, u):
            for j in range(GROUP // 128):
                pltpu.async_copy(hh_head.at[sdb[u % 4].at[j]],
                                 rowsb[u % 2].at[pl.ds(j * 128, 128)], sem_g)

        def wait_g(g, u):
            for j in range(GROUP // 128):
                pltpu.make_async_copy(
                    hh_head.at[sdb[u % 4].at[j]],
                    rowsb[u % 2].at[pl.ds(j * 128, 128)], sem_g).wait()

        def fire_s(u):
            for j in range(GROUP // 128):
                pltpu.async_copy(msgb[u % 2].at[pl.ds(j * 128, 128)],
                                 acc_v.at[sdb[u % 4].at[2 + j]], sem_s,
                                 add=True)

        def wait_s(u):
            for j in range(GROUP // 128):
                pltpu.make_async_copy(msgb[u % 2].at[pl.ds(j * 128, 128)],
                                      acc_v.at[sdb[u % 4].at[2 + j]],
                                      sem_s).wait()

        def sbody(u):
            def inner(t, carry2):
                base = t * 16
                w16 = wb[u % 4][pl.ds(base, 16)]
                for i in range(16):
                    r = base + i
                    wv = w16[i]
                    msgb[u % 2][r, pl.ds(0, 16)] = (
                        rowsb[u % 2][r, pl.ds(0, 16)] * wv)
                    msgb[u % 2][r, pl.ds(16, 16)] = (
                        rowsb[u % 2][r, pl.ds(16, 16)] * wv)
                plsc.store_scatter(
                    msgb[u % 2],
                    [base + iota16, jnp.full((16,), HEAD_DIM, jnp.int32)],
                    w16)
                return carry2
            lax.fori_loop(0, GROUP // 16, inner, 0)

        fire_sdw(0, 0)
        fire_sdw(1, 1)
        wait_sdw(0, 0)
        fire_g(0, 0)

        def outer(g0, carry):
            for u in range(4):
                g = g0 * 4 + u

                @pl.when(g <= ngroups - 2)
                def _():
                    wait_sdw(g + 1, u + 1)
                    fire_g(g + 1, u + 1)
                wait_g(g, u)
                sbody(u)

                @pl.when(g >= 1)
                def _():
                    wait_s(u - 1)
                fire_s(u)

                @pl.when(g <= ngroups - 3)
                def _():
                    fire_sdw(g + 2, u + 2)
            return carry
        lax.fori_loop(0, ngroups // 4, outer, 0)
        wait_s((ngroups - 1) % 4)
        plsc.subcore_barrier()
        pltpu.sync_copy(acc_v.at[pl.ds(tid * stripe, stripe)],
                        out_hbm.at[head].at[pl.ds(tid * stripe, stripe)])
        plsc.subcore_barrier()


def _sc_edge(sd, hh, w_all):
    N = hh.shape[1]
    k = pl.kernel(
        _edge_body,
        out_type=jax.ShapeDtypeStruct((HEADS, N, ACCW), jnp.float32),
        mesh=_MESH,
        scratch_types=(
            [pltpu.VMEM_SHARED((N, ACCW), jnp.float32)]
            + [pltpu.VMEM((4, 128), jnp.int32) for _ in range(4)]
            + [pltpu.VMEM((GROUP,), jnp.float32) for _ in range(4)]
            + [pltpu.VMEM((GROUP, HEAD_DIM), jnp.float32) for _ in range(2)]
            + [pltpu.VMEM((GROUP, ACCW), jnp.float32) for _ in range(2)]
            + [pltpu.SemaphoreType.DMA for _ in range(3)]
        ),
        compiler_params=_SC_PARAMS_UNTILED,
    )
    return k(sd, hh, w_all)


def kernel(user_ids, item_ids, text_features, image_features, edge_index, params):
    p = params
    bsz = user_ids.shape[0]

    half = HIDDEN // 2
    M_u = p['ua']['Wv'] @ p['ua']['Wo']
    c_u = p['ua']['bv'] @ p['ua']['Wo'] + p['ua']['bo']
    M_i = p['ia']['Wv'] @ p['ia']['Wo']
    c_i = p['ia']['bv'] @ p['ia']['Wo'] + p['ia']['bo']
    W_t = jnp.concatenate([p['ut_W'] @ M_u[:half], p['it_W'] @ M_i[:half]], axis=1)
    W_im = jnp.concatenate([p['ui_W'] @ M_u[half:], p['ii_W'] @ M_i[half:]], axis=1)
    bias = jnp.concatenate([
        p['ut_b'] @ M_u[:half] + p['ui_b'] @ M_u[half:] + c_u,
        p['it_b'] @ M_i[:half] + p['ii_b'] @ M_i[half:] + c_i,
    ], axis=0)

    P = _feat_matmul(text_features, image_features, W_t, W_im, bias)
    UM_all = P[:, :HIDDEN]
    IM_all = P[:, HIDDEN:]

    x_emb, x_mm = _sc_gather(user_ids, item_ids, p['user_table'],
                             p['item_table'], UM_all, IM_all)

    src = edge_index[0]
    dst = edge_index[1]
    E = src.shape[0]
    sd = jnp.concatenate([src.reshape(-1, 2, 128), dst.reshape(-1, 2, 128)],
                         axis=1)

    x = None
    for i in range(NUM_LAYERS):
        g = p['gat%d' % i]
        attm = jnp.concatenate([
            jax.scipy.linalg.block_diag(*[g['att_src'][k][:, None] for k in range(HEADS)]),
            jax.scipy.linalg.block_diag(*[g['att_dst'][k][:, None] for k in range(HEADS)]),
        ], axis=1)
        if i == 0:
            W = g['W']
            hh, a8 = _layer_pre0(x_emb, x_mm, W[:HIDDEN], W[HIDDEN:], attm)
        else:
            hh, a8 = _layer_pre(x, p['res%d_W' % (i - 1)], p['res%d_b' % (i - 1)],
                                g['W'], attm)
        a_t = a8.T
        w_all = _sc_w(sd, a_t[:HEADS].reshape(HEADS, 1, -1),
                      a_t[HEADS:].reshape(HEADS, 1, -1), E)
        acc = _sc_edge(sd, hh, w_all)
        x = _finalize(acc, g['b'], p['ln%d_g' % i], p['ln%d_b' % i])

    ue = x[:bsz]
    ie = x[bsz:2 * bsz]
    return _scores(ue, ie), ue, ie
